# Initial kernel scaffold; baseline (speedup 1.0000x reference)
import jax
import jax.numpy as jnp
from jax.experimental import pallas as pl


def kernel(x, weights):
    raise NotImplementedError("write your pallas kernel here")


# speedup vs baseline: 1.5751x; 1.5751x over previous
import functools

import jax
import jax.numpy as jnp
from jax import lax
from jax.experimental import pallas as pl
from jax.experimental.pallas import tpu as pltpu
from jax.experimental.pallas import tpu_sc as plsc

_DIM = 32
_BATCH = 16384
_N_FIELDS = 26
_B = _BATCH * _N_FIELDS
_NC, _NS = 2, 16
_NW = _NC * _NS
_BPW = _B // _NW
_C = 1024
_NCHUNK = _BPW // _C

_mesh = plsc.VectorSubcoreMesh(
    core_axis_name="c", subcore_axis_name="s", num_cores=_NC, num_subcores=_NS
)


@functools.partial(
    pl.kernel,
    out_type=jax.ShapeDtypeStruct((_B, _DIM), jnp.float32),
    mesh=_mesh,
    scratch_types=[
        pltpu.VMEM((_BPW,), jnp.int32),
        pltpu.VMEM((_C, _DIM), jnp.float32),
        pltpu.VMEM((_C, _DIM), jnp.float32),
        pltpu.SemaphoreType.DMA,
        pltpu.SemaphoreType.DMA,
        pltpu.SemaphoreType.DMA,
        pltpu.SemaphoreType.DMA,
    ],
    compiler_params=pltpu.CompilerParams(use_tc_tiling_on_sc=False),
)
def _gather_kernel(idx_hbm, table_hbm, out_hbm, idx_v, buf0, buf1,
                   sg0, sg1, ss0, ss1):
    wid = lax.axis_index("s") * _NC + lax.axis_index("c")
    base = wid * _BPW
    pltpu.sync_copy(idx_hbm.at[pl.ds(base, _BPW)], idx_v)
    bufs = (buf0, buf1)
    gsems = (sg0, sg1)
    ssems = (ss0, ss1)
    gathers = [None] * _NCHUNK
    stores = [None] * _NCHUNK
    gathers[0] = pltpu.async_copy(
        table_hbm.at[idx_v.at[pl.ds(0, _C)]], bufs[0], gsems[0])
    for g in range(_NCHUNK):
        nxt = g + 1
        if nxt < _NCHUNK:
            if nxt >= 2:
                stores[nxt - 2].wait()
            gathers[nxt] = pltpu.async_copy(
                table_hbm.at[idx_v.at[pl.ds(nxt * _C, _C)]],
                bufs[nxt % 2], gsems[nxt % 2])
        gathers[g].wait()
        stores[g] = pltpu.async_copy(
            bufs[g % 2], out_hbm.at[pl.ds(base + g * _C, _C)], ssems[g % 2])
    stores[_NCHUNK - 2].wait()
    stores[_NCHUNK - 1].wait()


def kernel(x, weights):
    idx = x.reshape(_B)
    out = _gather_kernel(idx, weights)
    return out.reshape(_BATCH, _N_FIELDS, _DIM)
